# baseline (speedup 1.0000x reference)
import functools
import math

import jax
import jax.numpy as jnp
from jax import lax
from jax.experimental import pallas as pl
from jax.experimental.pallas import tpu as pltpu
from jax.experimental.pallas import tpu_sc as plsc

NC = 2
NS = 16
NW = NC * NS
D = 128
CHUNK = 128
BR = 1000
EPS = 1e-5
_SQRT2 = math.sqrt(2.0)


@functools.partial(jax.jit, static_argnames=("n_nodes", "n_chunks"))
def _sc_seg_sum(h, src3, dst3, zeros, *, n_nodes, n_chunks):
    rows_per_tile = (-(-n_nodes // NS) + 7) // 8 * 8
    n_pad = NS * rows_per_tile
    mesh = plsc.VectorSubcoreMesh(
        core_axis_name="c", subcore_axis_name="s", num_cores=NC, num_subcores=NS
    )

    @functools.partial(
        pl.kernel,
        out_type=jax.ShapeDtypeStruct((NC, n_pad, D), jnp.float32),
        mesh=mesh,
        scratch_types=[
            pltpu.VMEM((4, CHUNK), jnp.int32),
            pltpu.VMEM((4, CHUNK), jnp.int32),
            pltpu.VMEM((2, CHUNK, D), jnp.float32),
            pltpu.VMEM_SHARED((n_pad + 16, D), jnp.float32),
            pltpu.SemaphoreType.DMA,
            pltpu.SemaphoreType.DMA,
            pltpu.SemaphoreType.DMA,
            pltpu.SemaphoreType.DMA,
            pltpu.SemaphoreType.DMA,
            pltpu.SemaphoreType.DMA,
        ],
    )
    def seg_sum(h_hbm, src_hbm, dst_hbm, zeros_hbm, out_hbm,
                srcv, dstv, rows_v, acc_sh,
                semr0, semr1, semi0, semi1, semi2, semi3):
        c = lax.axis_index("c")
        s = lax.axis_index("s")
        wid = s * NC + c
        row0 = s * rows_per_tile
        semr = (semr0, semr1)
        semi = (semi0, semi1, semi2, semi3)

        def idx_issue(j, sl):
            pltpu.async_copy(src_hbm.at[wid, j], srcv.at[sl], semi[sl])
            pltpu.async_copy(dst_hbm.at[wid, j], dstv.at[sl], semi[sl])

        def idx_wait(sl):
            pltpu.make_async_copy(src_hbm.at[wid, 0], srcv.at[sl],
                                  semi[sl]).wait()
            pltpu.make_async_copy(dst_hbm.at[wid, 0], dstv.at[sl],
                                  semi[sl]).wait()

        def gat_issue(sl, b):
            pltpu.async_copy(h_hbm.at[srcv.at[sl]], rows_v.at[b], semr[b])

        def gat_wait(b):
            pltpu.make_async_copy(h_hbm.at[srcv.at[0]], rows_v.at[b],
                                  semr[b]).wait()

        def scat(sl, b):
            pltpu.sync_copy(rows_v.at[b], acc_sh.at[dstv.at[sl]], add=True)

        pltpu.sync_copy(zeros_hbm.at[pl.ds(row0, rows_per_tile)],
                        acc_sh.at[pl.ds(row0, rows_per_tile)])
        for k in range(4):
            idx_issue(k, k)
        plsc.subcore_barrier()
        for k in range(2):
            idx_wait(k)
            gat_issue(k, k)

        def body(j2, carry):
            for k in range(4):
                j = j2 * 4 + k
                gat_wait(k % 2)
                scat(k, k % 2)

                @pl.when(j + 4 < n_chunks)
                def _():
                    idx_issue(j + 4, k)

                @pl.when(j + 2 < n_chunks)
                def _():
                    idx_wait((k + 2) % 4)
                    gat_issue((k + 2) % 4, k % 2)
            return carry

        lax.fori_loop(0, n_chunks // 4, body, 0)

        plsc.subcore_barrier()
        pltpu.sync_copy(acc_sh.at[pl.ds(row0, rows_per_tile)],
                        out_hbm.at[c, pl.ds(row0, rows_per_tile)])

    return seg_sum(h, src3, dst3, zeros)


_PREC = lax.Precision.HIGHEST


def _mlp_a(h, partials, w1, b1):
    n = h.shape[0]
    nb = n // BR

    def body(h_ref, pp_ref, w_ref, b_ref, u_ref):
        t = h_ref[...] + (pp_ref[0] + pp_ref[1])
        u_ref[...] = (
            jnp.dot(t, w_ref[...], preferred_element_type=jnp.float32) + b_ref[...]
        )

    return pl.pallas_call(
        body,
        grid=(nb,),
        in_specs=[
            pl.BlockSpec((BR, D), lambda i: (i, 0)),
            pl.BlockSpec((NC, BR, D), lambda i: (0, i, 0)),
            pl.BlockSpec((D, D), lambda i: (0, 0)),
            pl.BlockSpec((1, D), lambda i: (0, 0)),
        ],
        out_specs=pl.BlockSpec((BR, D), lambda i: (i, 0)),
        out_shape=jax.ShapeDtypeStruct((n, D), jnp.float32),
    )(h, partials, w1, b1.reshape(1, D))


def _mlp_stats(u):
    n = u.shape[0]

    def body(u_ref, m_ref, v_ref):
        uu = u_ref[...]
        m = jnp.sum(uu, axis=0, keepdims=True) / n
        m_ref[...] = m
        du = uu - m
        v_ref[...] = jnp.sum(du * du, axis=0, keepdims=True) / n

    return pl.pallas_call(
        body,
        out_shape=[
            jax.ShapeDtypeStruct((1, D), jnp.float32),
            jax.ShapeDtypeStruct((1, D), jnp.float32),
        ],
    )(u)


def _mlp_b(u, m_, v_, g, be, w2, b2):
    n = u.shape[0]
    nb = n // BR

    def body(u_ref, m_ref, v_ref, g_ref, be_ref, w_ref, b_ref, o_ref):
        r = jnp.maximum(
            (u_ref[...] - m_ref[...]) * lax.rsqrt(v_ref[...] + EPS) * g_ref[...]
            + be_ref[...], 0.0)
        z = jnp.dot(r, w_ref[...], preferred_element_type=jnp.float32) + b_ref[...]
        o_ref[...] = z * 0.5 * (1.0 + lax.erf(z / _SQRT2))

    return pl.pallas_call(
        body,
        grid=(nb,),
        in_specs=[
            pl.BlockSpec((BR, D), lambda i: (i, 0)),
            pl.BlockSpec((1, D), lambda i: (0, 0)),
            pl.BlockSpec((1, D), lambda i: (0, 0)),
            pl.BlockSpec((1, D), lambda i: (0, 0)),
            pl.BlockSpec((1, D), lambda i: (0, 0)),
            pl.BlockSpec((D, D), lambda i: (0, 0)),
            pl.BlockSpec((1, D), lambda i: (0, 0)),
        ],
        out_specs=pl.BlockSpec((BR, D), lambda i: (i, 0)),
        out_shape=jax.ShapeDtypeStruct((n, D), jnp.float32),
    )(u, m_, v_, g.reshape(1, D), be.reshape(1, D), w2, b2.reshape(1, D))


def _pool(h, batch3, n_graphs):
    n = h.shape[0]
    nb = n // BR

    def body(h_ref, b_ref, o_ref):
        bi = b_ref[0, 0, :]
        onehot_t = (
            lax.broadcasted_iota(jnp.int32, (n_graphs, BR), 0) == bi[None, :]
        ).astype(jnp.float32)
        acc = jnp.dot(onehot_t, h_ref[...], preferred_element_type=jnp.float32,
                      precision=_PREC)

        @pl.when(pl.program_id(0) == 0)
        def _():
            o_ref[...] = jnp.zeros_like(o_ref)

        o_ref[...] += acc

    return pl.pallas_call(
        body,
        grid=(nb,),
        in_specs=[
            pl.BlockSpec((BR, D), lambda i: (i, 0)),
            pl.BlockSpec((1, 1, BR), lambda i: (i, 0, 0)),
        ],
        out_specs=pl.BlockSpec((n_graphs, D), lambda i: (0, 0)),
        out_shape=jax.ShapeDtypeStruct((n_graphs, D), jnp.float32),
    )(h, batch3)


def _final_mlp(pooled, f):
    g_rows = pooled.shape[0]

    def body(p_ref, w1_ref, b1_ref, g_ref, be_ref, w2_ref, b2_ref, o_ref):
        u = jnp.dot(p_ref[...], w1_ref[...], preferred_element_type=jnp.float32)
        u = u + b1_ref[...]
        m = jnp.sum(u, axis=0, keepdims=True) / g_rows
        du = u - m
        v = jnp.sum(du * du, axis=0, keepdims=True) / g_rows
        r = jnp.maximum(du * lax.rsqrt(v + EPS) * g_ref[...] + be_ref[...], 0.0)
        o_ref[...] = (
            jnp.dot(r, w2_ref[...], preferred_element_type=jnp.float32)
            + b2_ref[...]
        )

    return pl.pallas_call(
        body,
        out_shape=jax.ShapeDtypeStruct((g_rows, D), jnp.float32),
    )(
        pooled,
        f["W1"],
        f["b1"].reshape(1, D),
        f["g1"].reshape(1, D),
        f["be1"].reshape(1, D),
        f["W2"],
        f["b2"].reshape(1, D),
    )


def kernel(x, edge_index, batch, batch_size, params):
    n = x.shape[0]
    e = edge_index.shape[1]
    n_graphs = 128

    n_pad = NS * ((-(-n // NS) + 7) // 8 * 8)
    need = int(e / NW + 12.0 * math.sqrt(e / NW + 1.0)) + 1
    n_chunks = max(4, -(-(-(-need // CHUNK)) // 4) * 4)
    cap = n_chunks * CHUNK
    src = edge_index[0]
    dst = edge_index[1]
    pos = jnp.arange(e, dtype=jnp.int32)
    perm1 = jnp.argsort(dst, stable=True)
    ds = dst[perm1]
    nstart = jnp.searchsorted(ds, jnp.arange(n, dtype=ds.dtype))
    rank = pos - nstart[ds].astype(jnp.int32)
    key2 = (ds.astype(jnp.int32) % NW) * 1024 + jnp.minimum(rank, 1023)
    skey, order = lax.sort((key2, perm1.astype(jnp.int32)), num_keys=1)
    sb = src[order]
    db = dst[order]
    bb = skey // 1024
    bounds = jnp.searchsorted(bb, jnp.arange(NW + 1, dtype=bb.dtype))
    idx2 = bounds[:NW, None] + jnp.arange(cap, dtype=jnp.int32)[None, :]
    valid = idx2 < bounds[1:, None]
    idx2c = jnp.minimum(idx2, e - 1)
    n_fill = max(1, n_pad - n)
    filler = n + (jnp.arange(cap, dtype=jnp.int32) % n_fill)
    src3 = jnp.where(valid, sb[idx2c], 0).reshape(NW, n_chunks, CHUNK)
    dst3 = jnp.where(valid, db[idx2c], filler[None, :]).reshape(
        NW, n_chunks, CHUNK)
    zeros = jnp.zeros((n_pad, D), jnp.float32)
    batch3 = batch.reshape(n // BR, 1, BR)

    h = x
    for p in params["convs"]:
        partials = _sc_seg_sum(h, src3, dst3, zeros,
                               n_nodes=n, n_chunks=n_chunks)
        u = _mlp_a(h, partials, p["W1"], p["b1"])
        m_, v_ = _mlp_stats(u)
        h = _mlp_b(u, m_, v_, p["g1"], p["be1"], p["W2"], p["b2"])

    pooled = _pool(h, batch3, n_graphs)
    return _final_mlp(pooled, params["final"])


# speedup vs baseline: 1.4169x; 1.4169x over previous
import functools
import math

import jax
import jax.numpy as jnp
from jax import lax
from jax.experimental import pallas as pl
from jax.experimental.pallas import tpu as pltpu
from jax.experimental.pallas import tpu_sc as plsc

NC = 2
NS = 16
NW = NC * NS
D = 128
CHUNK = 128
BR = 1000
EPS = 1e-5
_SQRT2 = math.sqrt(2.0)


@functools.partial(jax.jit, static_argnames=("n_nodes", "n_chunks"))
def _sc_seg_sum(h, src3, dst3, zeros, *, n_nodes, n_chunks):
    rows_per_tile = (-(-n_nodes // NS) + 7) // 8 * 8
    n_pad = NS * rows_per_tile
    mesh = plsc.VectorSubcoreMesh(
        core_axis_name="c", subcore_axis_name="s", num_cores=NC, num_subcores=NS
    )

    @functools.partial(
        pl.kernel,
        out_type=jax.ShapeDtypeStruct((NC, n_pad, D), jnp.float32),
        mesh=mesh,
        scratch_types=[
            pltpu.VMEM((4, CHUNK), jnp.int32),
            pltpu.VMEM((4, CHUNK), jnp.int32),
            pltpu.VMEM((2, CHUNK, D), jnp.float32),
            pltpu.VMEM_SHARED((n_pad + 16, D), jnp.float32),
            pltpu.SemaphoreType.DMA,
            pltpu.SemaphoreType.DMA,
            pltpu.SemaphoreType.DMA,
            pltpu.SemaphoreType.DMA,
            pltpu.SemaphoreType.DMA,
            pltpu.SemaphoreType.DMA,
        ],
    )
    def seg_sum(h_hbm, src_hbm, dst_hbm, zeros_hbm, out_hbm,
                srcv, dstv, rows_v, acc_sh,
                semr0, semr1, semi0, semi1, semi2, semi3):
        c = lax.axis_index("c")
        s = lax.axis_index("s")
        wid = s * NC + c
        row0 = s * rows_per_tile
        semr = (semr0, semr1)
        semi = (semi0, semi1, semi2, semi3)

        def idx_issue(j, sl):
            pltpu.async_copy(src_hbm.at[wid, j], srcv.at[sl], semi[sl])
            pltpu.async_copy(dst_hbm.at[wid, j], dstv.at[sl], semi[sl])

        def idx_wait(sl):
            pltpu.make_async_copy(src_hbm.at[wid, 0], srcv.at[sl],
                                  semi[sl]).wait()
            pltpu.make_async_copy(dst_hbm.at[wid, 0], dstv.at[sl],
                                  semi[sl]).wait()

        def gat_issue(sl, b):
            pltpu.async_copy(h_hbm.at[srcv.at[sl]], rows_v.at[b], semr[b])

        def gat_wait(b):
            pltpu.make_async_copy(h_hbm.at[srcv.at[0]], rows_v.at[b],
                                  semr[b]).wait()

        def scat(sl, b):
            pltpu.sync_copy(rows_v.at[b], acc_sh.at[dstv.at[sl]], add=True)

        pltpu.sync_copy(zeros_hbm.at[pl.ds(row0, rows_per_tile)],
                        acc_sh.at[pl.ds(row0, rows_per_tile)])
        for k in range(4):
            idx_issue(k, k)
        plsc.subcore_barrier()
        for k in range(2):
            idx_wait(k)
            gat_issue(k, k)

        def body(j2, carry):
            for k in range(4):
                j = j2 * 4 + k
                gat_wait(k % 2)
                scat(k, k % 2)

                @pl.when(j + 4 < n_chunks)
                def _():
                    idx_issue(j + 4, k)

                @pl.when(j + 2 < n_chunks)
                def _():
                    idx_wait((k + 2) % 4)
                    gat_issue((k + 2) % 4, k % 2)
            return carry

        lax.fori_loop(0, n_chunks // 4, body, 0)

        plsc.subcore_barrier()
        pltpu.sync_copy(acc_sh.at[pl.ds(row0, rows_per_tile)],
                        out_hbm.at[c, pl.ds(row0, rows_per_tile)])

    return seg_sum(h, src3, dst3, zeros)


_PREC = lax.Precision.HIGHEST


def _mlp_a(h, partials, w1, b1):
    n = h.shape[0]
    nb = n // BR

    def body(h_ref, pp_ref, w_ref, b_ref, u_ref):
        t = h_ref[...] + (pp_ref[0] + pp_ref[1])
        u_ref[...] = (
            jnp.dot(t, w_ref[...], preferred_element_type=jnp.float32) + b_ref[...]
        )

    return pl.pallas_call(
        body,
        grid=(nb,),
        in_specs=[
            pl.BlockSpec((BR, D), lambda i: (i, 0)),
            pl.BlockSpec((NC, BR, D), lambda i: (0, i, 0)),
            pl.BlockSpec((D, D), lambda i: (0, 0)),
            pl.BlockSpec((1, D), lambda i: (0, 0)),
        ],
        out_specs=pl.BlockSpec((BR, D), lambda i: (i, 0)),
        out_shape=jax.ShapeDtypeStruct((n, D), jnp.float32),
    )(h, partials, w1, b1.reshape(1, D))


def _mlp_stats(u):
    n = u.shape[0]

    def body(u_ref, m_ref, v_ref):
        uu = u_ref[...]
        m = jnp.sum(uu, axis=0, keepdims=True) / n
        m_ref[...] = m
        du = uu - m
        v_ref[...] = jnp.sum(du * du, axis=0, keepdims=True) / n

    return pl.pallas_call(
        body,
        out_shape=[
            jax.ShapeDtypeStruct((1, D), jnp.float32),
            jax.ShapeDtypeStruct((1, D), jnp.float32),
        ],
    )(u)


def _mlp_b(u, m_, v_, g, be, w2, b2):
    n = u.shape[0]
    nb = n // BR

    def body(u_ref, m_ref, v_ref, g_ref, be_ref, w_ref, b_ref, o_ref):
        r = jnp.maximum(
            (u_ref[...] - m_ref[...]) * lax.rsqrt(v_ref[...] + EPS) * g_ref[...]
            + be_ref[...], 0.0)
        z = jnp.dot(r, w_ref[...], preferred_element_type=jnp.float32) + b_ref[...]
        o_ref[...] = z * 0.5 * (1.0 + lax.erf(z / _SQRT2))

    return pl.pallas_call(
        body,
        grid=(nb,),
        in_specs=[
            pl.BlockSpec((BR, D), lambda i: (i, 0)),
            pl.BlockSpec((1, D), lambda i: (0, 0)),
            pl.BlockSpec((1, D), lambda i: (0, 0)),
            pl.BlockSpec((1, D), lambda i: (0, 0)),
            pl.BlockSpec((1, D), lambda i: (0, 0)),
            pl.BlockSpec((D, D), lambda i: (0, 0)),
            pl.BlockSpec((1, D), lambda i: (0, 0)),
        ],
        out_specs=pl.BlockSpec((BR, D), lambda i: (i, 0)),
        out_shape=jax.ShapeDtypeStruct((n, D), jnp.float32),
    )(u, m_, v_, g.reshape(1, D), be.reshape(1, D), w2, b2.reshape(1, D))


def _pool(h, batch3, n_graphs):
    n = h.shape[0]
    nb = n // BR

    def body(h_ref, b_ref, o_ref):
        bi = b_ref[0, 0, :]
        onehot_t = (
            lax.broadcasted_iota(jnp.int32, (n_graphs, BR), 0) == bi[None, :]
        ).astype(jnp.float32)
        acc = jnp.dot(onehot_t, h_ref[...], preferred_element_type=jnp.float32,
                      precision=_PREC)

        @pl.when(pl.program_id(0) == 0)
        def _():
            o_ref[...] = jnp.zeros_like(o_ref)

        o_ref[...] += acc

    return pl.pallas_call(
        body,
        grid=(nb,),
        in_specs=[
            pl.BlockSpec((BR, D), lambda i: (i, 0)),
            pl.BlockSpec((1, 1, BR), lambda i: (i, 0, 0)),
        ],
        out_specs=pl.BlockSpec((n_graphs, D), lambda i: (0, 0)),
        out_shape=jax.ShapeDtypeStruct((n_graphs, D), jnp.float32),
    )(h, batch3)


def _final_mlp(pooled, f):
    g_rows = pooled.shape[0]

    def body(p_ref, w1_ref, b1_ref, g_ref, be_ref, w2_ref, b2_ref, o_ref):
        u = jnp.dot(p_ref[...], w1_ref[...], preferred_element_type=jnp.float32)
        u = u + b1_ref[...]
        m = jnp.sum(u, axis=0, keepdims=True) / g_rows
        du = u - m
        v = jnp.sum(du * du, axis=0, keepdims=True) / g_rows
        r = jnp.maximum(du * lax.rsqrt(v + EPS) * g_ref[...] + be_ref[...], 0.0)
        o_ref[...] = (
            jnp.dot(r, w2_ref[...], preferred_element_type=jnp.float32)
            + b2_ref[...]
        )

    return pl.pallas_call(
        body,
        out_shape=jax.ShapeDtypeStruct((g_rows, D), jnp.float32),
    )(
        pooled,
        f["W1"],
        f["b1"].reshape(1, D),
        f["g1"].reshape(1, D),
        f["be1"].reshape(1, D),
        f["W2"],
        f["b2"].reshape(1, D),
    )


def kernel(x, edge_index, batch, batch_size, params):
    n = x.shape[0]
    e = edge_index.shape[1]
    n_graphs = 128

    n_pad = NS * ((-(-n // NS) + 7) // 8 * 8)
    need = int(e / NW + 12.0 * math.sqrt(e / NW + 1.0)) + 1
    n_chunks = max(4, -(-(-(-need // CHUNK)) // 4) * 4)
    cap = n_chunks * CHUNK
    src = edge_index[0]
    dst = edge_index[1]
    slices = 64
    rows_per_bucket = -(-n_pad // NW)
    slice_len = -(-e // slices)
    pos = jnp.arange(e, dtype=jnp.int32)
    key = (dst // rows_per_bucket) * slices + pos // slice_len
    val = src * 16384 + dst
    skey, sval = lax.sort((key, val), num_keys=1, is_stable=True)
    sb = sval // 16384
    db = sval % 16384
    bb = skey // slices
    bounds = jnp.searchsorted(bb, jnp.arange(NW + 1, dtype=bb.dtype))
    idx2 = bounds[:NW, None] + jnp.arange(cap, dtype=jnp.int32)[None, :]
    valid = idx2 < bounds[1:, None]
    idx2c = jnp.minimum(idx2, e - 1)
    n_fill = max(1, n_pad - n)
    filler = n + (jnp.arange(cap, dtype=jnp.int32) % n_fill)
    src3 = jnp.where(valid, sb[idx2c], 0).reshape(NW, n_chunks, CHUNK)
    dst3 = jnp.where(valid, db[idx2c], filler[None, :]).reshape(
        NW, n_chunks, CHUNK)
    zeros = jnp.zeros((n_pad, D), jnp.float32)
    batch3 = batch.reshape(n // BR, 1, BR)

    h = x
    for p in params["convs"]:
        partials = _sc_seg_sum(h, src3, dst3, zeros,
                               n_nodes=n, n_chunks=n_chunks)
        u = _mlp_a(h, partials, p["W1"], p["b1"])
        m_, v_ = _mlp_stats(u)
        h = _mlp_b(u, m_, v_, p["g1"], p["be1"], p["W2"], p["b2"])

    pooled = _pool(h, batch3, n_graphs)
    return _final_mlp(pooled, params["final"])


# speedup vs baseline: 7.7418x; 5.4640x over previous
import functools
import math

import jax
import jax.numpy as jnp
from jax import lax
from jax.experimental import pallas as pl
from jax.experimental.pallas import tpu as pltpu
from jax.experimental.pallas import tpu_sc as plsc

NC = 2
NS = 16
NW = NC * NS
D = 128
CHUNK = 128
BR = 1000
EPS = 1e-5
_SQRT2 = math.sqrt(2.0)


@functools.partial(jax.jit, static_argnames=("n_nodes", "n_chunks"))
def _sc_seg_sum(h, src3, dst3, zeros, *, n_nodes, n_chunks):
    rows_per_tile = (-(-n_nodes // NS) + 7) // 8 * 8
    n_pad = NS * rows_per_tile
    mesh = plsc.VectorSubcoreMesh(
        core_axis_name="c", subcore_axis_name="s", num_cores=NC, num_subcores=NS
    )

    @functools.partial(
        pl.kernel,
        out_type=jax.ShapeDtypeStruct((NC, n_pad, D), jnp.float32),
        mesh=mesh,
        scratch_types=[
            pltpu.VMEM((4, CHUNK), jnp.int32),
            pltpu.VMEM((4, CHUNK), jnp.int32),
            pltpu.VMEM((2, CHUNK, D), jnp.float32),
            pltpu.VMEM_SHARED((n_pad + 16, D), jnp.float32),
            pltpu.SemaphoreType.DMA,
            pltpu.SemaphoreType.DMA,
            pltpu.SemaphoreType.DMA,
            pltpu.SemaphoreType.DMA,
            pltpu.SemaphoreType.DMA,
            pltpu.SemaphoreType.DMA,
        ],
    )
    def seg_sum(h_hbm, src_hbm, dst_hbm, zeros_hbm, out_hbm,
                srcv, dstv, rows_v, acc_sh,
                semr0, semr1, semi0, semi1, semi2, semi3):
        c = lax.axis_index("c")
        s = lax.axis_index("s")
        wid = s * NC + c
        row0 = s * rows_per_tile
        semr = (semr0, semr1)
        semi = (semi0, semi1, semi2, semi3)

        def idx_issue(j, sl):
            pltpu.async_copy(src_hbm.at[wid, j], srcv.at[sl], semi[sl])
            pltpu.async_copy(dst_hbm.at[wid, j], dstv.at[sl], semi[sl])

        def idx_wait(sl):
            pltpu.make_async_copy(src_hbm.at[wid, 0], srcv.at[sl],
                                  semi[sl]).wait()
            pltpu.make_async_copy(dst_hbm.at[wid, 0], dstv.at[sl],
                                  semi[sl]).wait()

        def gat_issue(sl, b):
            pltpu.async_copy(h_hbm.at[srcv.at[sl]], rows_v.at[b], semr[b])

        def gat_wait(b):
            pltpu.make_async_copy(h_hbm.at[srcv.at[0]], rows_v.at[b],
                                  semr[b]).wait()

        def scat(sl, b):
            pltpu.sync_copy(rows_v.at[b], acc_sh.at[dstv.at[sl]], add=True)

        pltpu.sync_copy(zeros_hbm.at[pl.ds(row0, rows_per_tile)],
                        acc_sh.at[pl.ds(row0, rows_per_tile)])
        for k in range(4):
            idx_issue(k, k)
        plsc.subcore_barrier()
        for k in range(2):
            idx_wait(k)
            gat_issue(k, k)

        def body(j2, carry):
            for k in range(4):
                j = j2 * 4 + k
                gat_wait(k % 2)
                scat(k, k % 2)

                @pl.when(j + 4 < n_chunks)
                def _():
                    idx_issue(j + 4, k)

                @pl.when(j + 2 < n_chunks)
                def _():
                    idx_wait((k + 2) % 4)
                    gat_issue((k + 2) % 4, k % 2)
            return carry

        lax.fori_loop(0, n_chunks // 4, body, 0)

        plsc.subcore_barrier()
        pltpu.sync_copy(acc_sh.at[pl.ds(row0, rows_per_tile)],
                        out_hbm.at[c, pl.ds(row0, rows_per_tile)])

    return seg_sum(h, src3, dst3, zeros)


_PREC = lax.Precision.HIGHEST


def _mlp_a(h, partials, w1, b1):
    n = h.shape[0]
    nb = n // BR

    def body(h_ref, pp_ref, w_ref, b_ref, u_ref):
        t = h_ref[...] + (pp_ref[0] + pp_ref[1])
        u_ref[...] = (
            jnp.dot(t, w_ref[...], preferred_element_type=jnp.float32) + b_ref[...]
        )

    return pl.pallas_call(
        body,
        grid=(nb,),
        in_specs=[
            pl.BlockSpec((BR, D), lambda i: (i, 0)),
            pl.BlockSpec((NC, BR, D), lambda i: (0, i, 0)),
            pl.BlockSpec((D, D), lambda i: (0, 0)),
            pl.BlockSpec((1, D), lambda i: (0, 0)),
        ],
        out_specs=pl.BlockSpec((BR, D), lambda i: (i, 0)),
        out_shape=jax.ShapeDtypeStruct((n, D), jnp.float32),
    )(h, partials, w1, b1.reshape(1, D))


def _mlp_stats(u):
    n = u.shape[0]

    def body(u_ref, m_ref, v_ref):
        uu = u_ref[...]
        m = jnp.sum(uu, axis=0, keepdims=True) / n
        m_ref[...] = m
        du = uu - m
        v_ref[...] = jnp.sum(du * du, axis=0, keepdims=True) / n

    return pl.pallas_call(
        body,
        out_shape=[
            jax.ShapeDtypeStruct((1, D), jnp.float32),
            jax.ShapeDtypeStruct((1, D), jnp.float32),
        ],
    )(u)


def _mlp_b(u, m_, v_, g, be, w2, b2):
    n = u.shape[0]
    nb = n // BR

    def body(u_ref, m_ref, v_ref, g_ref, be_ref, w_ref, b_ref, o_ref):
        r = jnp.maximum(
            (u_ref[...] - m_ref[...]) * lax.rsqrt(v_ref[...] + EPS) * g_ref[...]
            + be_ref[...], 0.0)
        z = jnp.dot(r, w_ref[...], preferred_element_type=jnp.float32) + b_ref[...]
        o_ref[...] = z * 0.5 * (1.0 + lax.erf(z / _SQRT2))

    return pl.pallas_call(
        body,
        grid=(nb,),
        in_specs=[
            pl.BlockSpec((BR, D), lambda i: (i, 0)),
            pl.BlockSpec((1, D), lambda i: (0, 0)),
            pl.BlockSpec((1, D), lambda i: (0, 0)),
            pl.BlockSpec((1, D), lambda i: (0, 0)),
            pl.BlockSpec((1, D), lambda i: (0, 0)),
            pl.BlockSpec((D, D), lambda i: (0, 0)),
            pl.BlockSpec((1, D), lambda i: (0, 0)),
        ],
        out_specs=pl.BlockSpec((BR, D), lambda i: (i, 0)),
        out_shape=jax.ShapeDtypeStruct((n, D), jnp.float32),
    )(u, m_, v_, g.reshape(1, D), be.reshape(1, D), w2, b2.reshape(1, D))


def _pool(h, batch3, n_graphs):
    n = h.shape[0]
    nb = n // BR

    def body(h_ref, b_ref, o_ref):
        bi = b_ref[0, 0, :]
        onehot_t = (
            lax.broadcasted_iota(jnp.int32, (n_graphs, BR), 0) == bi[None, :]
        ).astype(jnp.float32)
        acc = jnp.dot(onehot_t, h_ref[...], preferred_element_type=jnp.float32,
                      precision=_PREC)

        @pl.when(pl.program_id(0) == 0)
        def _():
            o_ref[...] = jnp.zeros_like(o_ref)

        o_ref[...] += acc

    return pl.pallas_call(
        body,
        grid=(nb,),
        in_specs=[
            pl.BlockSpec((BR, D), lambda i: (i, 0)),
            pl.BlockSpec((1, 1, BR), lambda i: (i, 0, 0)),
        ],
        out_specs=pl.BlockSpec((n_graphs, D), lambda i: (0, 0)),
        out_shape=jax.ShapeDtypeStruct((n_graphs, D), jnp.float32),
    )(h, batch3)


def _final_mlp(pooled, f):
    g_rows = pooled.shape[0]

    def body(p_ref, w1_ref, b1_ref, g_ref, be_ref, w2_ref, b2_ref, o_ref):
        u = jnp.dot(p_ref[...], w1_ref[...], preferred_element_type=jnp.float32)
        u = u + b1_ref[...]
        m = jnp.sum(u, axis=0, keepdims=True) / g_rows
        du = u - m
        v = jnp.sum(du * du, axis=0, keepdims=True) / g_rows
        r = jnp.maximum(du * lax.rsqrt(v + EPS) * g_ref[...] + be_ref[...], 0.0)
        o_ref[...] = (
            jnp.dot(r, w2_ref[...], preferred_element_type=jnp.float32)
            + b2_ref[...]
        )

    return pl.pallas_call(
        body,
        out_shape=jax.ShapeDtypeStruct((g_rows, D), jnp.float32),
    )(
        pooled,
        f["W1"],
        f["b1"].reshape(1, D),
        f["g1"].reshape(1, D),
        f["be1"].reshape(1, D),
        f["W2"],
        f["b2"].reshape(1, D),
    )


def kernel(x, edge_index, batch, batch_size, params):
    n = x.shape[0]
    e = edge_index.shape[1]
    n_graphs = 128

    n_pad = NS * ((-(-n // NS) + 7) // 8 * 8)
    need = int(e / NW + 12.0 * math.sqrt(e / NW + 1.0)) + 1
    n_chunks = max(4, -(-(-(-need // CHUNK)) // 4) * 4)
    cap = n_chunks * CHUNK
    src = edge_index[0]
    dst = edge_index[1]
    slices = 64
    rows_per_bucket = -(-n_pad // NW)
    slice_len = -(-e // slices)
    pos = jnp.arange(e, dtype=jnp.int32)
    key = (dst // rows_per_bucket) * slices + pos // slice_len
    val = src * 16384 + dst
    skey, sval = lax.sort((key, val), num_keys=1, is_stable=True)
    sb = sval // 16384
    db = sval % 16384
    bb = skey // slices
    bounds = jnp.searchsorted(bb, jnp.arange(NW + 1, dtype=bb.dtype))
    idx2 = bounds[:NW, None] + jnp.arange(cap, dtype=jnp.int32)[None, :]
    valid = idx2 < bounds[1:, None]
    idx2c = jnp.minimum(idx2, e - 1)
    n_fill = max(1, n_pad - n)
    filler = n + (jnp.arange(cap, dtype=jnp.int32) % n_fill)
    src3 = jnp.where(valid, sb[idx2c], filler - n).reshape(NW, n_chunks, CHUNK)
    dst3 = jnp.where(valid, db[idx2c], filler[None, :]).reshape(
        NW, n_chunks, CHUNK)
    zeros = jnp.zeros((n_pad, D), jnp.float32)
    batch3 = batch.reshape(n // BR, 1, BR)

    h = x
    for p in params["convs"]:
        partials = _sc_seg_sum(h, src3, dst3, zeros,
                               n_nodes=n, n_chunks=n_chunks)
        u = _mlp_a(h, partials, p["W1"], p["b1"])
        m_, v_ = _mlp_stats(u)
        h = _mlp_b(u, m_, v_, p["g1"], p["be1"], p["W2"], p["b2"])

    pooled = _pool(h, batch3, n_graphs)
    return _final_mlp(pooled, params["final"])
